# Initial kernel scaffold; baseline (speedup 1.0000x reference)
import jax
import jax.numpy as jnp
from jax.experimental import pallas as pl


def kernel(x, edge_index, batch, W1_rel, W1_root, b1, W2_rel, W2_root, b2, W3_rel, W3_root, b3, W4_rel, W4_root, b4, fc_W, fc_b, reg_W, reg_b):
    raise NotImplementedError("write your pallas kernel here")


# speedup vs baseline: 2.4989x; 2.4989x over previous
import functools
import jax
import jax.numpy as jnp
from jax import lax
from jax.experimental import pallas as pl
from jax.experimental.pallas import tpu as pltpu
from jax.experimental.pallas import tpu_sc as plsc

_N = 10000
_E = 320000
_D = 128
_B = 16
_NW = 32
_CH = 128
_NCHUNK = 80
_NCPAD = _NCHUNK + 4
_EP = _NW * _NCHUNK * _CH
_NP = 10112
_RPS = _NP // 16

def _sc_aggregate_body(y_hbm, src_hbm, dst_hbm, zero_hbm, out_hbm,
                       si0, si1, di0, di1, buf0, buf1, aggr,
                       gsem0, gsem1, isem0, isem1):
    cid = lax.axis_index("c")
    sid = lax.axis_index("s")
    wid = sid * 2 + cid
    r0 = sid * _RPS

    pltpu.sync_copy(zero_hbm.at[pl.ds(r0, _RPS)], aggr.at[pl.ds(r0, _RPS)])
    plsc.subcore_barrier()

    pltpu.sync_copy(src_hbm.at[wid].at[0], si0)
    pltpu.sync_copy(dst_hbm.at[wid].at[0], di0)
    pltpu.async_copy(y_hbm.at[si0], buf0, gsem0)
    pltpu.async_copy(src_hbm.at[wid].at[1], si1, isem0)
    pltpu.async_copy(dst_hbm.at[wid].at[1], di1, isem1)

    def body(p, carry):
        j0 = 2 * p
        pltpu.make_async_copy(src_hbm.at[wid].at[j0 + 1], si1, isem0).wait()
        pltpu.make_async_copy(dst_hbm.at[wid].at[j0 + 1], di1, isem1).wait()
        pltpu.make_async_copy(y_hbm.at[si0], buf0, gsem0).wait()
        pltpu.async_copy(y_hbm.at[si1], buf1, gsem1)
        pltpu.sync_copy(buf0, aggr.at[di0], add=True)
        pltpu.async_copy(src_hbm.at[wid].at[j0 + 2], si0, isem0)
        pltpu.async_copy(dst_hbm.at[wid].at[j0 + 2], di0, isem1)
        pltpu.make_async_copy(src_hbm.at[wid].at[j0 + 2], si0, isem0).wait()
        pltpu.make_async_copy(dst_hbm.at[wid].at[j0 + 2], di0, isem1).wait()
        pltpu.make_async_copy(y_hbm.at[si1], buf1, gsem1).wait()
        pltpu.async_copy(y_hbm.at[si0], buf0, gsem0)
        pltpu.sync_copy(buf1, aggr.at[di1], add=True)
        pltpu.async_copy(src_hbm.at[wid].at[j0 + 3], si1, isem0)
        pltpu.async_copy(dst_hbm.at[wid].at[j0 + 3], di1, isem1)
        return carry

    lax.fori_loop(0, _NCHUNK // 2, body, 0)
    pltpu.make_async_copy(y_hbm.at[si0], buf0, gsem0).wait()
    pltpu.make_async_copy(src_hbm.at[wid].at[0], si1, isem0).wait()
    pltpu.make_async_copy(dst_hbm.at[wid].at[0], di1, isem1).wait()

    plsc.subcore_barrier()
    pltpu.sync_copy(aggr.at[pl.ds(r0, _RPS)],
                    out_hbm.at[cid].at[pl.ds(r0, _RPS)])


@functools.cache
def _sc_aggregate_call():
    mesh = plsc.VectorSubcoreMesh(core_axis_name="c", subcore_axis_name="s")
    return pl.kernel(
        _sc_aggregate_body,
        out_type=jax.ShapeDtypeStruct((2, _NP, _D), jnp.float32),
        mesh=mesh,
        scratch_types=[
            pltpu.VMEM((_CH,), jnp.int32),
            pltpu.VMEM((_CH,), jnp.int32),
            pltpu.VMEM((_CH,), jnp.int32),
            pltpu.VMEM((_CH,), jnp.int32),
            pltpu.VMEM((_CH, _D), jnp.float32),
            pltpu.VMEM((_CH, _D), jnp.float32),
            pltpu.VMEM_SHARED((_NP, _D), jnp.float32),
            pltpu.SemaphoreType.DMA,
            pltpu.SemaphoreType.DMA,
            pltpu.SemaphoreType.DMA,
            pltpu.SemaphoreType.DMA,
        ],
    )


_RB = 1000


def _pre_body(h_ref, wr_ref, ww_ref, b_ref, y_ref, z_ref):
    h = h_ref[...]
    y_ref[...] = jnp.dot(h, wr_ref[...], preferred_element_type=jnp.float32, precision=jax.lax.Precision.HIGHEST)
    z_ref[...] = jnp.dot(h, ww_ref[...], preferred_element_type=jnp.float32, precision=jax.lax.Precision.HIGHEST) + b_ref[...]


def _mid_body(p_ref, z_ref, wr_ref, ww_ref, b_ref, y_ref, z2_ref):
    h = jnp.maximum(p_ref[0] + p_ref[1] + z_ref[...], 0.0)
    y_ref[...] = jnp.dot(h, wr_ref[...], preferred_element_type=jnp.float32, precision=jax.lax.Precision.HIGHEST)
    z2_ref[...] = jnp.dot(h, ww_ref[...], preferred_element_type=jnp.float32, precision=jax.lax.Precision.HIGHEST) + b_ref[...]


def _post_body(p_ref, z_ref, bt_ref, fcw_ref, fcb_ref, rw_ref, rb_ref,
               out_ref, sums, cnts):
    i = pl.program_id(0)

    @pl.when(i == 0)
    def _init():
        sums[...] = jnp.zeros_like(sums)
        cnts[...] = jnp.zeros_like(cnts)

    h = jnp.maximum(p_ref[0] + p_ref[1] + z_ref[...], 0.0)
    bt = bt_ref[0, 0, :]
    mask = (bt[None, :] == lax.broadcasted_iota(jnp.int32, (_B, _RB), 0)
            ).astype(jnp.float32)
    sums[...] += jnp.dot(mask, h, preferred_element_type=jnp.float32, precision=jax.lax.Precision.HIGHEST)
    cnts[...] += jnp.sum(mask, axis=1, keepdims=True)

    @pl.when(i == pl.num_programs(0) - 1)
    def _finish():
        pooled = sums[...] / jnp.maximum(cnts[...], 1.0)
        fc = jnp.dot(pooled, fcw_ref[...],
                     preferred_element_type=jnp.float32, precision=jax.lax.Precision.HIGHEST) + fcb_ref[...]
        o = jnp.dot(fc, rw_ref[...],
                    preferred_element_type=jnp.float32, precision=jax.lax.Precision.HIGHEST) + rb_ref[...]
        out_ref[...] = jnp.broadcast_to(o, (_B, _D))


_row_spec = pl.BlockSpec((_RB, _D), lambda i: (i, 0))
_p_spec = pl.BlockSpec((2, _RB, _D), lambda i: (0, i, 0))
_w_spec = pl.BlockSpec((_D, _D), lambda i: (0, 0))
_b_spec = pl.BlockSpec((1, _D), lambda i: (0, 0))

_pre_call = pl.pallas_call(
    _pre_body,
    grid=(_N // _RB,),
    in_specs=[_row_spec, _w_spec, _w_spec, _b_spec],
    out_specs=[_row_spec, _row_spec],
    out_shape=[jax.ShapeDtypeStruct((_N, _D), jnp.float32)] * 2,
)

_mid_call = pl.pallas_call(
    _mid_body,
    grid=(_N // _RB,),
    in_specs=[_p_spec, _row_spec, _w_spec, _w_spec, _b_spec],
    out_specs=[_row_spec, _row_spec],
    out_shape=[jax.ShapeDtypeStruct((_N, _D), jnp.float32)] * 2,
)

_post_call = pl.pallas_call(
    _post_body,
    grid=(_N // _RB,),
    in_specs=[
        _p_spec,
        _row_spec,
        pl.BlockSpec((1, 1, _RB), lambda i: (i, 0, 0)),
        _w_spec,
        _b_spec,
        pl.BlockSpec((_D, 1), lambda i: (0, 0)),
        pl.BlockSpec((1, 1), lambda i: (0, 0)),
    ],
    out_specs=pl.BlockSpec((_B, _D), lambda i: (0, 0)),
    out_shape=jax.ShapeDtypeStruct((_B, _D), jnp.float32),
    scratch_shapes=[
        pltpu.VMEM((_B, _D), jnp.float32),
        pltpu.VMEM((_B, _D), jnp.float32),
    ],
)


def kernel(x, edge_index, batch, W1_rel, W1_root, b1, W2_rel, W2_root, b2,
           W3_rel, W3_root, b3, W4_rel, W4_root, b4, fc_W, fc_b, reg_W, reg_b):
    src = edge_index[0]
    dst = edge_index[1]
    pad = _EP - _E
    tail = jnp.zeros((_NW, _NCPAD - _NCHUNK, _CH), jnp.int32)
    srcp = jnp.concatenate(
        [src, jnp.zeros((pad,), jnp.int32)]).reshape(_NW, _NCHUNK, _CH)
    srcp = jnp.concatenate([srcp, tail], axis=1)
    dstp = jnp.concatenate(
        [dst, jnp.full((pad,), _N, jnp.int32)]).reshape(_NW, _NCHUNK, _CH)
    dstp = jnp.concatenate([dstp, tail], axis=1)
    zero_init = jnp.zeros((_NP, _D), jnp.float32)
    bt = batch.reshape(_N // _RB, 1, _RB)

    y, z = _pre_call(x, W1_rel, W1_root, b1.reshape(1, _D))
    for (wr, ww, b) in ((W2_rel, W2_root, b2), (W3_rel, W3_root, b3),
                        (W4_rel, W4_root, b4)):
        p = _sc_aggregate_call()(y, srcp, dstp, zero_init)
        y, z = _mid_call(p, z, wr, ww, b.reshape(1, _D))
    p = _sc_aggregate_call()(y, srcp, dstp, zero_init)
    out = _post_call(p, z, bt, fc_W, fc_b.reshape(1, _D), reg_W,
                     reg_b.reshape(1, 1))
    return out[:, :1]
